# baseline (speedup 1.0000x reference)
import functools

import jax
import jax.numpy as jnp
from jax import lax
from jax.experimental import pallas as pl
from jax.experimental.pallas import tpu as pltpu
from jax.experimental.pallas import tpu_sc as plsc

VAR_LEN = 1000000
EMBED = 64
B = 4096
S = 200
N = B * S

NC = 2
NS = 16
NW = NC * NS
PER_W = N // NW
CHUNK = 256
NCHUNK = PER_W // CHUNK
LANES = 16
VPR = EMBED // LANES


def _make_sc_kernel():
    mesh = plsc.VectorSubcoreMesh(core_axis_name="c", subcore_axis_name="s")

    @functools.partial(
        pl.kernel,
        mesh=mesh,
        out_type=jax.ShapeDtypeStruct((N, EMBED), jnp.float32),
        compiler_params=pltpu.CompilerParams(use_tc_tiling_on_sc=False),
        scratch_types=[
            pltpu.VMEM((CHUNK,), jnp.int32),
            pltpu.VMEM((CHUNK, EMBED), jnp.float32),
            pltpu.VMEM((CHUNK, EMBED), jnp.float32),
            pltpu.VMEM((CHUNK, EMBED), jnp.float32),
            pltpu.SemaphoreType.DMA,
        ],
    )
    def emb_kernel(x_h, idx_h, pos_h, tab_h, out_h, idxb, rowb, xb, pb, sem):
        wid = lax.axis_index("s") * NC + lax.axis_index("c")
        base = wid * PER_W

        def chunk_body(g, carry):
            b0 = base + g * CHUNK
            pltpu.sync_copy(idx_h.at[pl.ds(b0, CHUNK)], idxb)
            c_row = pltpu.async_copy(tab_h.at[idxb], rowb, sem)
            c_x = pltpu.async_copy(x_h.at[pl.ds(b0, CHUNK)], xb, sem)
            c_p = pltpu.async_copy(pos_h.at[pl.ds(b0, CHUNK)], pb, sem)
            c_row.wait()
            c_x.wait()
            c_p.wait()

            def row_body(r, carry2):
                for k in range(VPR):
                    sl = pl.ds(k * LANES, LANES)
                    xb[r, sl] = xb[r, sl] + pb[r, sl] + rowb[r, sl]
                return carry2

            lax.fori_loop(0, CHUNK, row_body, 0, unroll=2)
            pltpu.sync_copy(xb, out_h.at[pl.ds(b0, CHUNK)])
            return carry

        lax.fori_loop(0, NCHUNK, chunk_body, 0)

    return emb_kernel


_sc_kernel = _make_sc_kernel()


def kernel(x, variable_seq, pos_emb, table):
    xf = x.reshape(N, EMBED)
    pf = pos_emb.reshape(N, EMBED)
    idx = variable_seq.reshape(N).astype(jnp.int32)
    out = _sc_kernel(xf, idx, pf, table)
    return out.reshape(B, S, EMBED)


# speedup vs baseline: 1.1048x; 1.1048x over previous
import functools

import jax
import jax.numpy as jnp
from jax import lax
from jax.experimental import pallas as pl
from jax.experimental.pallas import tpu as pltpu
from jax.experimental.pallas import tpu_sc as plsc

VAR_LEN = 1000000
EMBED = 64
B = 4096
S = 200
N = B * S

NC = 2
NS = 16
NW = NC * NS
PER_W = N // NW
CHUNK = 200
NCHUNK = PER_W // CHUNK
LANES = 16
VPR = EMBED // LANES


def _make_sc_kernel():
    mesh = plsc.VectorSubcoreMesh(core_axis_name="c", subcore_axis_name="s")

    vbuf = lambda: pltpu.VMEM((CHUNK, EMBED), jnp.float32)

    @functools.partial(
        pl.kernel,
        mesh=mesh,
        out_type=jax.ShapeDtypeStruct((N, EMBED), jnp.float32),
        compiler_params=pltpu.CompilerParams(use_tc_tiling_on_sc=False),
        scratch_types=[
            pltpu.VMEM((CHUNK,), jnp.int32), pltpu.VMEM((CHUNK,), jnp.int32),
            vbuf(), vbuf(),
            vbuf(), vbuf(),
            vbuf(), vbuf(),
            vbuf(), vbuf(),
        ] + [pltpu.SemaphoreType.DMA] * 8,
    )
    def emb_kernel(x_h, idx_h, pos_h, tab_h, out_h,
                   idx0, idx1, xb0, xb1, pb0, pb1, rb0, rb1, ob0, ob1,
                   nsem0, nsem1, isem0, isem1, gsem0, gsem1, osem0, osem1):
        wid = lax.axis_index("s") * NC + lax.axis_index("c")
        base = wid * PER_W

        slots = ((idx0, xb0, pb0, rb0, ob0, nsem0, isem0, gsem0, osem0),
                 (idx1, xb1, pb1, rb1, ob1, nsem1, isem1, gsem1, osem1))

        def issue_idx(g, slot):
            idxb, nsem = slots[slot][0], slots[slot][5]
            pltpu.async_copy(idx_h.at[pl.ds(base + g * CHUNK, CHUNK)], idxb, nsem)

        def wait_idx(g, slot):
            idxb, nsem = slots[slot][0], slots[slot][5]
            pltpu.make_async_copy(
                idx_h.at[pl.ds(base + g * CHUNK, CHUNK)], idxb, nsem).wait()

        def issue_xp(g, slot):
            _, xb, pb, _, _, _, isem, _, _ = slots[slot]
            b0 = base + g * CHUNK
            pltpu.async_copy(x_h.at[pl.ds(b0, CHUNK)], xb, isem)
            pltpu.async_copy(pos_h.at[pl.ds(b0, CHUNK)], pb, isem)

        def wait_xp(g, slot):
            _, xb, pb, _, _, _, isem, _, _ = slots[slot]
            b0 = base + g * CHUNK
            pltpu.make_async_copy(x_h.at[pl.ds(b0, CHUNK)], xb, isem).wait()
            pltpu.make_async_copy(pos_h.at[pl.ds(b0, CHUNK)], pb, isem).wait()

        def issue_gather(g, slot):
            idxb, rb, gsem = slots[slot][0], slots[slot][3], slots[slot][7]
            pltpu.async_copy(tab_h.at[idxb], rb, gsem)

        def wait_gather(g, slot):
            idxb, rb, gsem = slots[slot][0], slots[slot][3], slots[slot][7]
            pltpu.make_async_copy(tab_h.at[idxb], rb, gsem).wait()

        def issue_out(g, slot):
            ob, osem = slots[slot][4], slots[slot][8]
            pltpu.async_copy(ob, out_h.at[pl.ds(base + g * CHUNK, CHUNK)], osem)

        def wait_out(g, slot):
            ob, osem = slots[slot][4], slots[slot][8]
            pltpu.make_async_copy(
                ob, out_h.at[pl.ds(base + g * CHUNK, CHUNK)], osem).wait()

        def compute(slot):
            _, xb, pb, rb, ob = slots[slot][:5]

            def row_body(r, carry):
                for k in range(VPR):
                    sl = pl.ds(k * LANES, LANES)
                    ob[r, sl] = xb[r, sl] + pb[r, sl] + rb[r, sl]
                return carry

            lax.fori_loop(0, CHUNK, row_body, 0, unroll=4)

        def process(g, slot):
            wait_xp(g, slot)
            wait_gather(g, slot)
            compute(slot)
            issue_out(g, slot)

            @pl.when(g + 2 < NCHUNK)
            def _():
                issue_idx(g + 2, slot)
                issue_xp(g + 2, slot)
                wait_idx(g + 2, slot)
                issue_gather(g + 2, slot)

        issue_idx(0, 0)
        issue_xp(0, 0)
        issue_idx(1, 1)
        issue_xp(1, 1)
        wait_idx(0, 0)
        issue_gather(0, 0)
        wait_idx(1, 1)
        issue_gather(1, 1)

        def pair_body(gg, carry):
            g0 = 2 * gg
            g1 = g0 + 1

            @pl.when(gg > 0)
            def _():
                wait_out(g0 - 2, 0)
                wait_out(g1 - 2, 1)

            process(g0, 0)
            process(g1, 1)
            return carry

        lax.fori_loop(0, NCHUNK // 2, pair_body, 0)
        wait_out(NCHUNK - 2, 0)
        wait_out(NCHUNK - 1, 1)

    return emb_kernel


_sc_kernel = _make_sc_kernel()


def kernel(x, variable_seq, pos_emb, table):
    xf = x.reshape(N, EMBED)
    pf = pos_emb.reshape(N, EMBED)
    idx = variable_seq.reshape(N).astype(jnp.int32)
    out = _sc_kernel(xf, idx, pf, table)
    return out.reshape(B, S, EMBED)


# speedup vs baseline: 1.3432x; 1.2158x over previous
import functools

import jax
import jax.numpy as jnp
from jax import lax
from jax.experimental import pallas as pl
from jax.experimental.pallas import tpu as pltpu
from jax.experimental.pallas import tpu_sc as plsc

VAR_LEN = 1000000
EMBED = 64
B = 4096
S = 200
N = B * S

NC = 2
NS = 16
NW = NC * NS
PER_W = N // NW
CHUNK = 800
NCHUNK = PER_W // CHUNK

BB = 512


def _make_sc_gather():
    mesh = plsc.VectorSubcoreMesh(core_axis_name="c", subcore_axis_name="s")

    @functools.partial(
        pl.kernel,
        mesh=mesh,
        out_type=jax.ShapeDtypeStruct((N, EMBED), jnp.float32),
        compiler_params=pltpu.CompilerParams(use_tc_tiling_on_sc=False),
        scratch_types=[
            pltpu.VMEM((CHUNK,), jnp.int32), pltpu.VMEM((CHUNK,), jnp.int32),
            pltpu.VMEM((CHUNK, EMBED), jnp.float32),
            pltpu.VMEM((CHUNK, EMBED), jnp.float32),
        ] + [pltpu.SemaphoreType.DMA] * 6,
    )
    def gather_kernel(idx_h, tab_h, out_h,
                      idx0, idx1, rb0, rb1,
                      nsem0, nsem1, gsem0, gsem1, osem0, osem1):
        wid = lax.axis_index("s") * NC + lax.axis_index("c")
        base = wid * PER_W

        slots = ((idx0, rb0, nsem0, gsem0, osem0),
                 (idx1, rb1, nsem1, gsem1, osem1))

        def issue_idx(g, slot):
            idxb, _, nsem, _, _ = slots[slot]
            pltpu.async_copy(idx_h.at[pl.ds(base + g * CHUNK, CHUNK)], idxb, nsem)

        def wait_idx(g, slot):
            idxb, _, nsem, _, _ = slots[slot]
            pltpu.make_async_copy(
                idx_h.at[pl.ds(base + g * CHUNK, CHUNK)], idxb, nsem).wait()

        def issue_gather(g, slot):
            idxb, rb, _, gsem, _ = slots[slot]
            pltpu.async_copy(tab_h.at[idxb], rb, gsem)

        def wait_gather(g, slot):
            idxb, rb, _, gsem, _ = slots[slot]
            pltpu.make_async_copy(tab_h.at[idxb], rb, gsem).wait()

        def issue_out(g, slot):
            _, rb, _, _, osem = slots[slot]
            pltpu.async_copy(rb, out_h.at[pl.ds(base + g * CHUNK, CHUNK)], osem)

        def wait_out(g, slot):
            _, rb, _, _, osem = slots[slot]
            pltpu.make_async_copy(
                rb, out_h.at[pl.ds(base + g * CHUNK, CHUNK)], osem).wait()

        issue_idx(0, 0)
        issue_idx(1, 1)
        wait_idx(0, 0)
        issue_gather(0, 0)
        wait_idx(1, 1)
        issue_gather(1, 1)

        def pair_body(gg, carry):
            g0 = 2 * gg
            g1 = g0 + 1

            def do(g, slot):
                wait_gather(g, slot)
                issue_out(g, slot)

                @pl.when(g + 2 < NCHUNK)
                def _():
                    issue_idx(g + 2, slot)
                    wait_out(g, slot)
                    wait_idx(g + 2, slot)
                    issue_gather(g + 2, slot)

                @pl.when(g + 2 >= NCHUNK)
                def _():
                    wait_out(g, slot)

            do(g0, 0)
            do(g1, 1)
            return carry

        lax.fori_loop(0, NCHUNK // 2, pair_body, 0)

    return gather_kernel


_sc_gather = _make_sc_gather()


def _tc_add_body(x_ref, p_ref, v_ref, o_ref):
    vt = jnp.transpose(v_ref[0], (1, 0))
    o_ref[0] = x_ref[0] + p_ref[0] + vt


def _make_tc_add():
    grid = (S, B // BB)
    return pl.pallas_call(
        _tc_add_body,
        grid=grid,
        in_specs=[
            pl.BlockSpec((1, EMBED, BB), lambda s, j: (s, 0, j)),
            pl.BlockSpec((1, EMBED, BB), lambda s, j: (s, 0, j)),
            pl.BlockSpec((1, BB, EMBED), lambda s, j: (s, j, 0)),
        ],
        out_specs=pl.BlockSpec((1, EMBED, BB), lambda s, j: (s, 0, j)),
        out_shape=jax.ShapeDtypeStruct((S, EMBED, B), jnp.float32),
    )


_tc_add = _make_tc_add()


def kernel(x, variable_seq, pos_emb, table):
    idx_f = jnp.transpose(variable_seq, (1, 0)).reshape(N).astype(jnp.int32)
    var2 = _sc_gather(idx_f, table)
    x_t = jnp.transpose(x, (1, 2, 0))
    p_t = jnp.transpose(pos_emb, (1, 2, 0))
    o_t = _tc_add(x_t, p_t, var2.reshape(S, B, EMBED))
    return jnp.transpose(o_t, (2, 0, 1))


# speedup vs baseline: 2.0699x; 1.5410x over previous
import functools

import jax
import jax.numpy as jnp
from jax import lax
from jax.experimental import pallas as pl
from jax.experimental.pallas import tpu as pltpu
from jax.experimental.pallas import tpu_sc as plsc

VAR_LEN = 1000000
EMBED = 64
B = 4096
S = 200
N = B * S

NC = 2
NS = 16
NW = NC * NS
PER_W = N // NW
CHUNK = 800
NCHUNK = PER_W // CHUNK

SBLK = 2


def _make_sc_gather():
    mesh = plsc.VectorSubcoreMesh(core_axis_name="c", subcore_axis_name="s")

    @functools.partial(
        pl.kernel,
        mesh=mesh,
        out_type=jax.ShapeDtypeStruct((N, EMBED), jnp.float32),
        compiler_params=pltpu.CompilerParams(use_tc_tiling_on_sc=False),
        scratch_types=[
            pltpu.VMEM((CHUNK,), jnp.int32), pltpu.VMEM((CHUNK,), jnp.int32),
            pltpu.VMEM((CHUNK, EMBED), jnp.float32),
            pltpu.VMEM((CHUNK, EMBED), jnp.float32),
        ] + [pltpu.SemaphoreType.DMA] * 6,
    )
    def gather_kernel(idx_h, tab_h, out_h,
                      idx0, idx1, rb0, rb1,
                      nsem0, nsem1, gsem0, gsem1, osem0, osem1):
        wid = lax.axis_index("s") * NC + lax.axis_index("c")
        base = wid * PER_W

        slots = ((idx0, rb0, nsem0, gsem0, osem0),
                 (idx1, rb1, nsem1, gsem1, osem1))

        def issue_idx(g, slot):
            idxb, _, nsem, _, _ = slots[slot]
            pltpu.async_copy(idx_h.at[pl.ds(base + g * CHUNK, CHUNK)], idxb, nsem)

        def wait_idx(g, slot):
            idxb, _, nsem, _, _ = slots[slot]
            pltpu.make_async_copy(
                idx_h.at[pl.ds(base + g * CHUNK, CHUNK)], idxb, nsem).wait()

        def issue_gather(g, slot):
            idxb, rb, _, gsem, _ = slots[slot]
            pltpu.async_copy(tab_h.at[idxb], rb, gsem)

        def wait_gather(g, slot):
            idxb, rb, _, gsem, _ = slots[slot]
            pltpu.make_async_copy(tab_h.at[idxb], rb, gsem).wait()

        def issue_out(g, slot):
            _, rb, _, _, osem = slots[slot]
            pltpu.async_copy(rb, out_h.at[pl.ds(base + g * CHUNK, CHUNK)], osem)

        def wait_out(g, slot):
            _, rb, _, _, osem = slots[slot]
            pltpu.make_async_copy(
                rb, out_h.at[pl.ds(base + g * CHUNK, CHUNK)], osem).wait()

        issue_idx(0, 0)
        issue_idx(1, 1)
        wait_idx(0, 0)
        issue_gather(0, 0)
        wait_idx(1, 1)
        issue_gather(1, 1)

        def pair_body(gg, carry):
            g0 = 2 * gg
            g1 = g0 + 1

            def do(g, slot):
                wait_gather(g, slot)
                issue_out(g, slot)

                @pl.when(g + 2 < NCHUNK)
                def _():
                    issue_idx(g + 2, slot)
                    wait_out(g, slot)
                    wait_idx(g + 2, slot)
                    issue_gather(g + 2, slot)

                @pl.when(g + 2 >= NCHUNK)
                def _():
                    wait_out(g, slot)

            do(g0, 0)
            do(g1, 1)
            return carry

        lax.fori_loop(0, NCHUNK // 2, pair_body, 0)

    return gather_kernel


_sc_gather = _make_sc_gather()


def _tc_add_body(x_ref, p_ref, v_ref, o_ref):
    for i in range(SBLK):
        vt = jnp.transpose(v_ref[i], (1, 0))
        o_ref[i] = x_ref[i] + p_ref[i] + vt


def _make_tc_add():
    grid = (S // SBLK,)
    return pl.pallas_call(
        _tc_add_body,
        grid=grid,
        in_specs=[
            pl.BlockSpec((SBLK, EMBED, B), lambda s: (s, 0, 0)),
            pl.BlockSpec((SBLK, EMBED, B), lambda s: (s, 0, 0)),
            pl.BlockSpec((SBLK, B, EMBED), lambda s: (s, 0, 0)),
        ],
        out_specs=pl.BlockSpec((SBLK, EMBED, B), lambda s: (s, 0, 0)),
        out_shape=jax.ShapeDtypeStruct((S, EMBED, B), jnp.float32),
    )


_tc_add = _make_tc_add()


def kernel(x, variable_seq, pos_emb, table):
    idx_f = jnp.transpose(variable_seq, (1, 0)).reshape(N).astype(jnp.int32)
    var2 = _sc_gather(idx_f, table)
    x_t = jnp.transpose(x, (1, 2, 0))
    p_t = jnp.transpose(pos_emb, (1, 2, 0))
    o_t = _tc_add(x_t, p_t, var2.reshape(S, B, EMBED))
    return jnp.transpose(o_t, (2, 0, 1))


# speedup vs baseline: 2.1924x; 1.0592x over previous
import functools

import jax
import jax.numpy as jnp
from jax import lax
from jax.experimental import pallas as pl
from jax.experimental.pallas import tpu as pltpu
from jax.experimental.pallas import tpu_sc as plsc

VAR_LEN = 1000000
EMBED = 64
B = 4096
S = 200
N = B * S

NC = 2
NS = 16
NW = NC * NS
PER_W = N // NW
CHUNK = 800
NCHUNK = PER_W // CHUNK

SBLK = 2

VBLK = 2048
VPAD = 489 * VBLK


def _make_sc_gather():
    mesh = plsc.VectorSubcoreMesh(core_axis_name="c", subcore_axis_name="s")

    @functools.partial(
        pl.kernel,
        mesh=mesh,
        out_type=jax.ShapeDtypeStruct((N, EMBED), jnp.float32),
        compiler_params=pltpu.CompilerParams(use_tc_tiling_on_sc=False),
        name="sc_embed_gather",
        scratch_types=[
            pltpu.VMEM((CHUNK,), jnp.int32), pltpu.VMEM((CHUNK,), jnp.int32),
            pltpu.VMEM((CHUNK, EMBED), jnp.float32),
            pltpu.VMEM((CHUNK, EMBED), jnp.float32),
        ] + [pltpu.SemaphoreType.DMA] * 6,
    )
    def gather_kernel(idx_h, tab_h, out_h,
                      idx0, idx1, rb0, rb1,
                      nsem0, nsem1, gsem0, gsem1, osem0, osem1):
        wid = lax.axis_index("s") * NC + lax.axis_index("c")
        base = wid * PER_W

        slots = ((idx0, rb0, nsem0, gsem0, osem0),
                 (idx1, rb1, nsem1, gsem1, osem1))

        def issue_idx(g, slot):
            idxb, _, nsem, _, _ = slots[slot]
            pltpu.async_copy(idx_h.at[pl.ds(base + g * CHUNK, CHUNK)], idxb, nsem)

        def wait_idx(g, slot):
            idxb, _, nsem, _, _ = slots[slot]
            pltpu.make_async_copy(
                idx_h.at[pl.ds(base + g * CHUNK, CHUNK)], idxb, nsem).wait()

        def issue_gather(g, slot):
            idxb, rb, _, gsem, _ = slots[slot]
            pltpu.async_copy(tab_h.at[idxb], rb, gsem)

        def wait_gather(g, slot):
            idxb, rb, _, gsem, _ = slots[slot]
            pltpu.make_async_copy(tab_h.at[idxb], rb, gsem).wait()

        def issue_out(g, slot):
            _, rb, _, _, osem = slots[slot]
            pltpu.async_copy(rb, out_h.at[pl.ds(base + g * CHUNK, CHUNK)], osem)

        def wait_out(g, slot):
            _, rb, _, _, osem = slots[slot]
            pltpu.make_async_copy(
                rb, out_h.at[pl.ds(base + g * CHUNK, CHUNK)], osem).wait()

        issue_idx(0, 0)
        issue_idx(1, 1)
        wait_idx(0, 0)
        issue_gather(0, 0)
        wait_idx(1, 1)
        issue_gather(1, 1)

        def pair_body(gg, carry):
            g0 = 2 * gg
            g1 = g0 + 1

            def do(g, slot):
                wait_gather(g, slot)
                issue_out(g, slot)

                @pl.when(g + 2 < NCHUNK)
                def _():
                    issue_idx(g + 2, slot)
                    wait_out(g, slot)
                    wait_idx(g + 2, slot)
                    issue_gather(g + 2, slot)

                @pl.when(g + 2 >= NCHUNK)
                def _():
                    wait_out(g, slot)

            do(g0, 0)
            do(g1, 1)
            return carry

        lax.fori_loop(0, NCHUNK // 2, pair_body, 0)

    return gather_kernel


_sc_gather = _make_sc_gather()


def _tab_prep_body(t_ref, o_ref):
    t = jnp.transpose(t_ref[...], (1, 0))
    o_ref[...] = jnp.concatenate([t, t], axis=1)


_tab_prep = pl.pallas_call(
    _tab_prep_body,
    grid=(VPAD // VBLK,),
    in_specs=[pl.BlockSpec((EMBED, VBLK), lambda j: (0, j))],
    out_specs=pl.BlockSpec((VBLK, 2 * EMBED), lambda j: (j, 0)),
    out_shape=jax.ShapeDtypeStruct((VPAD, 2 * EMBED), jnp.float32),
)


def _tc_add_body(x_ref, p_ref, v_ref, o_ref):
    for i in range(SBLK):
        vt = jnp.transpose(v_ref[i], (1, 0))
        o_ref[i] = x_ref[i] + p_ref[i] + vt


def _make_tc_add():
    grid = (S // SBLK,)
    return pl.pallas_call(
        _tc_add_body,
        grid=grid,
        in_specs=[
            pl.BlockSpec((SBLK, EMBED, B), lambda s: (s, 0, 0)),
            pl.BlockSpec((SBLK, EMBED, B), lambda s: (s, 0, 0)),
            pl.BlockSpec((SBLK, B, EMBED), lambda s: (s, 0, 0)),
        ],
        out_specs=pl.BlockSpec((SBLK, EMBED, B), lambda s: (s, 0, 0)),
        out_shape=jax.ShapeDtypeStruct((S, EMBED, B), jnp.float32),
    )


_tc_add = _make_tc_add()


def kernel(x, variable_seq, pos_emb, table):
    idx_f = jnp.transpose(variable_seq, (1, 0)).reshape(N).astype(jnp.int32)
    tab_t = jnp.transpose(table, (1, 0))
    tab2 = _tab_prep(tab_t).reshape(2 * VPAD, EMBED)
    var2 = _sc_gather(idx_f * 2, tab2)
    x_t = jnp.transpose(x, (1, 2, 0))
    p_t = jnp.transpose(pos_emb, (1, 2, 0))
    o_t = _tc_add(x_t, p_t, var2.reshape(S, B, EMBED))
    return jnp.transpose(o_t, (2, 0, 1))


# speedup vs baseline: 2.5077x; 1.1438x over previous
import functools

import jax
import jax.numpy as jnp
from jax import lax
from jax.experimental import pallas as pl
from jax.experimental.pallas import tpu as pltpu
from jax.experimental.pallas import tpu_sc as plsc

VAR_LEN = 1000000
EMBED = 64
B = 4096
S = 200
N = B * S

NC = 2
NS = 16
NW = NC * NS
PER_W = N // NW
CHUNK = 400
NCHUNK = PER_W // CHUNK

SBLK = 2

VBLK = 2048
VPAD = 489 * VBLK


def _make_sc_gather():
    mesh = plsc.VectorSubcoreMesh(core_axis_name="c", subcore_axis_name="s")

    @functools.partial(
        pl.kernel,
        mesh=mesh,
        out_type=jax.ShapeDtypeStruct((N, 2 * EMBED), jnp.float32),
        compiler_params=pltpu.CompilerParams(use_tc_tiling_on_sc=False),
        name="sc_embed_gather",
        scratch_types=[
            pltpu.VMEM((CHUNK,), jnp.int32), pltpu.VMEM((CHUNK,), jnp.int32),
            pltpu.VMEM((CHUNK, 2 * EMBED), jnp.float32),
            pltpu.VMEM((CHUNK, 2 * EMBED), jnp.float32),
        ] + [pltpu.SemaphoreType.DMA] * 6,
    )
    def gather_kernel(idx_h, tab_h, out_h,
                      idx0, idx1, rb0, rb1,
                      nsem0, nsem1, gsem0, gsem1, osem0, osem1):
        wid = lax.axis_index("s") * NC + lax.axis_index("c")
        base = wid * PER_W

        slots = ((idx0, rb0, nsem0, gsem0, osem0),
                 (idx1, rb1, nsem1, gsem1, osem1))

        def issue_idx(g, slot):
            idxb, _, nsem, _, _ = slots[slot]
            pltpu.async_copy(idx_h.at[pl.ds(base + g * CHUNK, CHUNK)], idxb, nsem)

        def wait_idx(g, slot):
            idxb, _, nsem, _, _ = slots[slot]
            pltpu.make_async_copy(
                idx_h.at[pl.ds(base + g * CHUNK, CHUNK)], idxb, nsem).wait()

        def issue_gather(g, slot):
            idxb, rb, _, gsem, _ = slots[slot]
            pltpu.async_copy(tab_h.at[idxb], rb, gsem)

        def wait_gather(g, slot):
            idxb, rb, _, gsem, _ = slots[slot]
            pltpu.make_async_copy(tab_h.at[idxb], rb, gsem).wait()

        def issue_out(g, slot):
            _, rb, _, _, osem = slots[slot]
            pltpu.async_copy(rb, out_h.at[pl.ds(base + g * CHUNK, CHUNK)], osem)

        def wait_out(g, slot):
            _, rb, _, _, osem = slots[slot]
            pltpu.make_async_copy(
                rb, out_h.at[pl.ds(base + g * CHUNK, CHUNK)], osem).wait()

        issue_idx(0, 0)
        issue_idx(1, 1)
        wait_idx(0, 0)
        issue_gather(0, 0)
        wait_idx(1, 1)
        issue_gather(1, 1)

        def pair_body(gg, carry):
            g0 = 2 * gg
            g1 = g0 + 1

            def do(g, slot):
                wait_gather(g, slot)
                issue_out(g, slot)

                @pl.when(g + 2 < NCHUNK)
                def _():
                    issue_idx(g + 2, slot)
                    wait_out(g, slot)
                    wait_idx(g + 2, slot)
                    issue_gather(g + 2, slot)

                @pl.when(g + 2 >= NCHUNK)
                def _():
                    wait_out(g, slot)

            do(g0, 0)
            do(g1, 1)
            return carry

        lax.fori_loop(0, NCHUNK // 2, pair_body, 0)

    return gather_kernel


_sc_gather = _make_sc_gather()


def _tab_prep_body(t_ref, o_ref):
    t = jnp.transpose(t_ref[...], (1, 0))
    o_ref[...] = jnp.concatenate([t, t], axis=1)


_tab_prep = pl.pallas_call(
    _tab_prep_body,
    grid=(VPAD // VBLK,),
    in_specs=[pl.BlockSpec((EMBED, VBLK), lambda j: (0, j))],
    out_specs=pl.BlockSpec((VBLK, 2 * EMBED), lambda j: (j, 0)),
    out_shape=jax.ShapeDtypeStruct((VPAD, 2 * EMBED), jnp.float32),
)


def _tc_add_body(x_ref, p_ref, v_ref, o_ref):
    for i in range(SBLK):
        vt = jnp.transpose(v_ref[i], (1, 0))
        o_ref[i] = x_ref[i] + p_ref[i] + vt[:EMBED]


def _make_tc_add():
    grid = (S // SBLK,)
    return pl.pallas_call(
        _tc_add_body,
        grid=grid,
        in_specs=[
            pl.BlockSpec((SBLK, EMBED, B), lambda s: (s, 0, 0)),
            pl.BlockSpec((SBLK, EMBED, B), lambda s: (s, 0, 0)),
            pl.BlockSpec((SBLK, B, 2 * EMBED), lambda s: (s, 0, 0)),
        ],
        out_specs=pl.BlockSpec((SBLK, EMBED, B), lambda s: (s, 0, 0)),
        out_shape=jax.ShapeDtypeStruct((S, EMBED, B), jnp.float32),
    )


_tc_add = _make_tc_add()


def kernel(x, variable_seq, pos_emb, table):
    idx_f = jnp.transpose(variable_seq, (1, 0)).reshape(N).astype(jnp.int32)
    tab_t = jnp.transpose(table, (1, 0))
    tab2 = _tab_prep(tab_t)
    var2 = _sc_gather(idx_f, tab2)
    x_t = jnp.transpose(x, (1, 2, 0))
    p_t = jnp.transpose(pos_emb, (1, 2, 0))
    o_t = _tc_add(x_t, p_t, var2.reshape(S, B, 2 * EMBED))
    return jnp.transpose(o_t, (2, 0, 1))


# speedup vs baseline: 2.6305x; 1.0489x over previous
import functools

import jax
import jax.numpy as jnp
from jax import lax
from jax.experimental import pallas as pl
from jax.experimental.pallas import tpu as pltpu
from jax.experimental.pallas import tpu_sc as plsc

VAR_LEN = 1000000
EMBED = 64
B = 4096
S = 200
N = B * S

NC = 2
NS = 16
NW = NC * NS
PER_W = N // NW
CHUNK = 400
NCHUNK = PER_W // CHUNK

SBLK = 2

VBLK = 2048
VPAD = 489 * VBLK


def _make_sc_gather():
    mesh = plsc.VectorSubcoreMesh(core_axis_name="c", subcore_axis_name="s")

    @functools.partial(
        pl.kernel,
        mesh=mesh,
        out_type=jax.ShapeDtypeStruct((N, 2 * EMBED), jnp.float32),
        compiler_params=pltpu.CompilerParams(use_tc_tiling_on_sc=False),
        name="sc_embed_gather",
        scratch_types=[
            pltpu.VMEM((CHUNK,), jnp.int32), pltpu.VMEM((CHUNK,), jnp.int32),
            pltpu.VMEM((CHUNK, 2 * EMBED), jnp.float32),
            pltpu.VMEM((CHUNK, 2 * EMBED), jnp.float32),
        ] + [pltpu.SemaphoreType.DMA] * 6,
    )
    def gather_kernel(idx_h, tab_h, out_h,
                      idx0, idx1, rb0, rb1,
                      nsem0, nsem1, gsem0, gsem1, osem0, osem1):
        wid = lax.axis_index("s") * NC + lax.axis_index("c")
        base = wid * PER_W

        slots = ((idx0, rb0, nsem0, gsem0, osem0),
                 (idx1, rb1, nsem1, gsem1, osem1))

        def issue_idx(g, slot):
            idxb, _, nsem, _, _ = slots[slot]
            pltpu.async_copy(idx_h.at[pl.ds(base + g * CHUNK, CHUNK)], idxb, nsem)

        def wait_idx(g, slot):
            idxb, _, nsem, _, _ = slots[slot]
            pltpu.make_async_copy(
                idx_h.at[pl.ds(base + g * CHUNK, CHUNK)], idxb, nsem).wait()

        def issue_gather(g, slot):
            idxb, rb, _, gsem, _ = slots[slot]
            pltpu.async_copy(tab_h.at[idxb], rb, gsem)

        def wait_gather(g, slot):
            idxb, rb, _, gsem, _ = slots[slot]
            pltpu.make_async_copy(tab_h.at[idxb], rb, gsem).wait()

        def issue_out(g, slot):
            _, rb, _, _, osem = slots[slot]
            pltpu.async_copy(rb, out_h.at[pl.ds(base + g * CHUNK, CHUNK)], osem)

        def wait_out(g, slot):
            _, rb, _, _, osem = slots[slot]
            pltpu.make_async_copy(
                rb, out_h.at[pl.ds(base + g * CHUNK, CHUNK)], osem).wait()

        issue_idx(0, 0)
        issue_idx(1, 1)
        wait_idx(0, 0)
        issue_gather(0, 0)
        wait_idx(1, 1)
        issue_gather(1, 1)

        def pair_body(gg, carry):
            g0 = 2 * gg
            g1 = g0 + 1

            def do(g, slot):
                wait_gather(g, slot)
                issue_out(g, slot)

                @pl.when(g + 2 < NCHUNK)
                def _():
                    issue_idx(g + 2, slot)
                    wait_out(g, slot)
                    wait_idx(g + 2, slot)
                    issue_gather(g + 2, slot)

                @pl.when(g + 2 >= NCHUNK)
                def _():
                    wait_out(g, slot)

            do(g0, 0)
            do(g1, 1)
            return carry

        lax.fori_loop(0, NCHUNK // 2, pair_body, 0)

    return gather_kernel


_sc_gather = _make_sc_gather()


def _tab_prep_body(t_ref, o_ref):
    t = jnp.transpose(t_ref[...], (1, 0))
    o_ref[:, :EMBED] = t


_tab_prep = pl.pallas_call(
    _tab_prep_body,
    grid=(VPAD // VBLK,),
    in_specs=[pl.BlockSpec((EMBED, VBLK), lambda j: (0, j))],
    out_specs=pl.BlockSpec((VBLK, 2 * EMBED), lambda j: (j, 0)),
    out_shape=jax.ShapeDtypeStruct((VPAD, 2 * EMBED), jnp.float32),
)


def _tc_add_body(x_ref, p_ref, v_ref, o_ref):
    for i in range(SBLK):
        vt = jnp.transpose(v_ref[i], (1, 0))
        o_ref[i] = x_ref[i] + p_ref[i] + vt[:EMBED]


def _make_tc_add():
    grid = (S // SBLK,)
    return pl.pallas_call(
        _tc_add_body,
        grid=grid,
        in_specs=[
            pl.BlockSpec((SBLK, EMBED, B), lambda s: (s, 0, 0)),
            pl.BlockSpec((SBLK, EMBED, B), lambda s: (s, 0, 0)),
            pl.BlockSpec((SBLK, B, 2 * EMBED), lambda s: (s, 0, 0)),
        ],
        out_specs=pl.BlockSpec((SBLK, EMBED, B), lambda s: (s, 0, 0)),
        out_shape=jax.ShapeDtypeStruct((S, EMBED, B), jnp.float32),
    )


_tc_add = _make_tc_add()


def kernel(x, variable_seq, pos_emb, table):
    idx_f = jnp.transpose(variable_seq, (1, 0)).reshape(N).astype(jnp.int32)
    tab_t = jnp.transpose(table, (1, 0))
    tab2 = _tab_prep(tab_t)
    var2 = _sc_gather(idx_f, tab2)
    x_t = jnp.transpose(x, (1, 2, 0))
    p_t = jnp.transpose(pos_emb, (1, 2, 0))
    o_t = _tc_add(x_t, p_t, var2.reshape(S, B, 2 * EMBED))
    return jnp.transpose(o_t, (2, 0, 1))


# speedup vs baseline: 3.1037x; 1.1799x over previous
import functools

import jax
import jax.numpy as jnp
from jax import lax
from jax.experimental import pallas as pl
from jax.experimental.pallas import tpu as pltpu
from jax.experimental.pallas import tpu_sc as plsc

VAR_LEN = 1000000
EMBED = 64
B = 4096
S = 200
N = B * S

NC = 2
NS = 16
NW = NC * NS
PER_W = N // NW
CHUNK = 400
NCHUNK = PER_W // CHUNK

SBLK = 2

VBLK = 4096
VPAD = 245 * VBLK


def _make_sc_gather():
    mesh = plsc.VectorSubcoreMesh(core_axis_name="c", subcore_axis_name="s")

    @functools.partial(
        pl.kernel,
        mesh=mesh,
        out_type=jax.ShapeDtypeStruct((N, 2 * EMBED), jnp.float32),
        compiler_params=pltpu.CompilerParams(use_tc_tiling_on_sc=False),
        name="sc_embed_gather",
        scratch_types=[
            pltpu.VMEM((CHUNK,), jnp.int32), pltpu.VMEM((CHUNK,), jnp.int32),
            pltpu.VMEM((CHUNK, 2 * EMBED), jnp.float32),
            pltpu.VMEM((CHUNK, 2 * EMBED), jnp.float32),
        ] + [pltpu.SemaphoreType.DMA] * 6,
    )
    def gather_kernel(idx_h, tab_h, out_h,
                      idx0, idx1, rb0, rb1,
                      nsem0, nsem1, gsem0, gsem1, osem0, osem1):
        wid = lax.axis_index("s") * NC + lax.axis_index("c")
        base = wid * PER_W

        slots = ((idx0, rb0, nsem0, gsem0, osem0),
                 (idx1, rb1, nsem1, gsem1, osem1))

        def issue_idx(g, slot):
            idxb, _, nsem, _, _ = slots[slot]
            pltpu.async_copy(idx_h.at[pl.ds(base + g * CHUNK, CHUNK)], idxb, nsem)

        def wait_idx(g, slot):
            idxb, _, nsem, _, _ = slots[slot]
            pltpu.make_async_copy(
                idx_h.at[pl.ds(base + g * CHUNK, CHUNK)], idxb, nsem).wait()

        def issue_gather(g, slot):
            idxb, rb, _, gsem, _ = slots[slot]
            pltpu.async_copy(tab_h.at[idxb], rb, gsem)

        def wait_gather(g, slot):
            idxb, rb, _, gsem, _ = slots[slot]
            pltpu.make_async_copy(tab_h.at[idxb], rb, gsem).wait()

        def issue_out(g, slot):
            _, rb, _, _, osem = slots[slot]
            pltpu.async_copy(
                rb.at[:, pl.ds(0, EMBED)],
                out_h.at[pl.ds(base + g * CHUNK, CHUNK), pl.ds(0, EMBED)],
                osem)

        def wait_out(g, slot):
            _, rb, _, _, osem = slots[slot]
            pltpu.make_async_copy(
                rb.at[:, pl.ds(0, EMBED)],
                out_h.at[pl.ds(base + g * CHUNK, CHUNK), pl.ds(0, EMBED)],
                osem).wait()

        issue_idx(0, 0)
        issue_idx(1, 1)
        wait_idx(0, 0)
        issue_gather(0, 0)
        wait_idx(1, 1)
        issue_gather(1, 1)

        def pair_body(gg, carry):
            g0 = 2 * gg
            g1 = g0 + 1

            def do(g, slot):
                wait_gather(g, slot)
                issue_out(g, slot)

                @pl.when(g + 2 < NCHUNK)
                def _():
                    issue_idx(g + 2, slot)
                    wait_out(g, slot)
                    wait_idx(g + 2, slot)
                    issue_gather(g + 2, slot)

                @pl.when(g + 2 >= NCHUNK)
                def _():
                    wait_out(g, slot)

            do(g0, 0)
            do(g1, 1)
            return carry

        lax.fori_loop(0, NCHUNK // 2, pair_body, 0)

    return gather_kernel


_sc_gather = _make_sc_gather()


def _tab_prep_body(t_ref, o_ref):
    t = jnp.transpose(t_ref[...], (1, 0))
    o_ref[:, :EMBED] = t


_tab_prep = pl.pallas_call(
    _tab_prep_body,
    grid=(VPAD // VBLK,),
    in_specs=[pl.BlockSpec((EMBED, VBLK), lambda j: (0, j))],
    out_specs=pl.BlockSpec((VBLK, 2 * EMBED), lambda j: (j, 0)),
    out_shape=jax.ShapeDtypeStruct((VPAD, 2 * EMBED), jnp.float32),
)


def _tc_add_body(x_ref, p_ref, v_ref, o_ref):
    for i in range(SBLK):
        vt = jnp.transpose(v_ref[i], (1, 0))
        o_ref[i] = x_ref[i] + p_ref[i] + vt[:EMBED]


def _make_tc_add():
    grid = (S // SBLK,)
    return pl.pallas_call(
        _tc_add_body,
        grid=grid,
        in_specs=[
            pl.BlockSpec((SBLK, EMBED, B), lambda s: (s, 0, 0)),
            pl.BlockSpec((SBLK, EMBED, B), lambda s: (s, 0, 0)),
            pl.BlockSpec((SBLK, B, 2 * EMBED), lambda s: (s, 0, 0)),
        ],
        out_specs=pl.BlockSpec((SBLK, EMBED, B), lambda s: (s, 0, 0)),
        out_shape=jax.ShapeDtypeStruct((S, EMBED, B), jnp.float32),
    )


_tc_add = _make_tc_add()


def kernel(x, variable_seq, pos_emb, table):
    idx_f = jnp.transpose(variable_seq, (1, 0)).reshape(N).astype(jnp.int32)
    tab_t = jnp.transpose(table, (1, 0))
    tab2 = _tab_prep(tab_t)
    var2 = _sc_gather(idx_f, tab2)
    x_t = jnp.transpose(x, (1, 2, 0))
    p_t = jnp.transpose(pos_emb, (1, 2, 0))
    o_t = _tc_add(x_t, p_t, var2.reshape(S, B, 2 * EMBED))
    return jnp.transpose(o_t, (2, 0, 1))


# speedup vs baseline: 3.2099x; 1.0342x over previous
import functools

import jax
import jax.numpy as jnp
from jax import lax
from jax.experimental import pallas as pl
from jax.experimental.pallas import tpu as pltpu
from jax.experimental.pallas import tpu_sc as plsc

VAR_LEN = 1000000
EMBED = 64
B = 4096
S = 200
N = B * S

NC = 2
NS = 16
NW = NC * NS
PER_W = N // NW
CHUNK = 400
NCHUNK = PER_W // CHUNK

SBLK = 2

VBLK = 8192
VPAD = 123 * VBLK


def _make_sc_gather():
    mesh = plsc.VectorSubcoreMesh(core_axis_name="c", subcore_axis_name="s")

    @functools.partial(
        pl.kernel,
        mesh=mesh,
        out_type=jax.ShapeDtypeStruct((N, 2 * EMBED), jnp.float32),
        compiler_params=pltpu.CompilerParams(use_tc_tiling_on_sc=False),
        name="sc_embed_gather",
        scratch_types=[
            pltpu.VMEM((CHUNK,), jnp.int32), pltpu.VMEM((CHUNK,), jnp.int32),
            pltpu.VMEM((CHUNK, 2 * EMBED), jnp.float32),
            pltpu.VMEM((CHUNK, 2 * EMBED), jnp.float32),
        ] + [pltpu.SemaphoreType.DMA] * 6,
    )
    def gather_kernel(idx_h, tab_h, out_h,
                      idx0, idx1, rb0, rb1,
                      nsem0, nsem1, gsem0, gsem1, osem0, osem1):
        wid = lax.axis_index("s") * NC + lax.axis_index("c")
        base = wid * PER_W

        slots = ((idx0, rb0, nsem0, gsem0, osem0),
                 (idx1, rb1, nsem1, gsem1, osem1))

        def issue_idx(g, slot):
            idxb, _, nsem, _, _ = slots[slot]
            pltpu.async_copy(idx_h.at[pl.ds(base + g * CHUNK, CHUNK)], idxb, nsem)

        def wait_idx(g, slot):
            idxb, _, nsem, _, _ = slots[slot]
            pltpu.make_async_copy(
                idx_h.at[pl.ds(base + g * CHUNK, CHUNK)], idxb, nsem).wait()

        def issue_gather(g, slot):
            idxb, rb, _, gsem, _ = slots[slot]
            pltpu.async_copy(tab_h.at[idxb], rb, gsem)

        def wait_gather(g, slot):
            idxb, rb, _, gsem, _ = slots[slot]
            pltpu.make_async_copy(tab_h.at[idxb], rb, gsem).wait()

        def issue_out(g, slot):
            _, rb, _, _, osem = slots[slot]
            pltpu.async_copy(rb, out_h.at[pl.ds(base + g * CHUNK, CHUNK)], osem)

        def wait_out(g, slot):
            _, rb, _, _, osem = slots[slot]
            pltpu.make_async_copy(
                rb, out_h.at[pl.ds(base + g * CHUNK, CHUNK)], osem).wait()

        issue_idx(0, 0)
        issue_idx(1, 1)
        wait_idx(0, 0)
        issue_gather(0, 0)
        wait_idx(1, 1)
        issue_gather(1, 1)

        def pair_body(gg, carry):
            g0 = 2 * gg
            g1 = g0 + 1

            def do(g, slot):
                wait_gather(g, slot)
                issue_out(g, slot)

                @pl.when(g + 2 < NCHUNK)
                def _():
                    issue_idx(g + 2, slot)
                    wait_out(g, slot)
                    wait_idx(g + 2, slot)
                    issue_gather(g + 2, slot)

                @pl.when(g + 2 >= NCHUNK)
                def _():
                    wait_out(g, slot)

            do(g0, 0)
            do(g1, 1)
            return carry

        lax.fori_loop(0, NCHUNK // 2, pair_body, 0)

    return gather_kernel


_sc_gather = _make_sc_gather()


def _tab_prep_body(t_ref, o_ref):
    t = jnp.transpose(t_ref[...], (1, 0))
    o_ref[:, :EMBED] = t


_tab_prep = pl.pallas_call(
    _tab_prep_body,
    grid=(VPAD // VBLK,),
    in_specs=[pl.BlockSpec((EMBED, VBLK), lambda j: (0, j))],
    out_specs=pl.BlockSpec((VBLK, 2 * EMBED), lambda j: (j, 0)),
    out_shape=jax.ShapeDtypeStruct((VPAD, 2 * EMBED), jnp.float32),
)


def _tc_add_body(x_ref, p_ref, v_ref, o_ref):
    for i in range(SBLK):
        vt = jnp.transpose(v_ref[i], (1, 0))
        o_ref[i] = x_ref[i] + p_ref[i] + vt[:EMBED]


def _make_tc_add():
    grid = (S // SBLK,)
    return pl.pallas_call(
        _tc_add_body,
        grid=grid,
        in_specs=[
            pl.BlockSpec((SBLK, EMBED, B), lambda s: (s, 0, 0)),
            pl.BlockSpec((SBLK, EMBED, B), lambda s: (s, 0, 0)),
            pl.BlockSpec((SBLK, B, 2 * EMBED), lambda s: (s, 0, 0)),
        ],
        out_specs=pl.BlockSpec((SBLK, EMBED, B), lambda s: (s, 0, 0)),
        out_shape=jax.ShapeDtypeStruct((S, EMBED, B), jnp.float32),
    )


_tc_add = _make_tc_add()


def kernel(x, variable_seq, pos_emb, table):
    idx_f = jnp.transpose(variable_seq, (1, 0)).reshape(N).astype(jnp.int32)
    tab_t = jnp.transpose(table, (1, 0))
    tab2 = _tab_prep(tab_t)
    var2 = _sc_gather(idx_f, tab2)
    x_t = jnp.transpose(x, (1, 2, 0))
    p_t = jnp.transpose(pos_emb, (1, 2, 0))
    o_t = _tc_add(x_t, p_t, var2.reshape(S, B, 2 * EMBED))
    return jnp.transpose(o_t, (2, 0, 1))


# speedup vs baseline: 3.2836x; 1.0230x over previous
import functools

import jax
import jax.numpy as jnp
from jax import lax
from jax.experimental import pallas as pl
from jax.experimental.pallas import tpu as pltpu
from jax.experimental.pallas import tpu_sc as plsc

VAR_LEN = 1000000
EMBED = 64
B = 4096
S = 200
N = B * S

NC = 2
NS = 16
NW = NC * NS
PER_W = N // NW
CHUNK = 400
NCHUNK = PER_W // CHUNK

SBLK = 4

VBLK = 16384
VPAD = 62 * VBLK


def _make_sc_gather():
    mesh = plsc.VectorSubcoreMesh(core_axis_name="c", subcore_axis_name="s")

    @functools.partial(
        pl.kernel,
        mesh=mesh,
        out_type=jax.ShapeDtypeStruct((N, 2 * EMBED), jnp.float32),
        compiler_params=pltpu.CompilerParams(use_tc_tiling_on_sc=False),
        name="sc_embed_gather",
        scratch_types=[
            pltpu.VMEM((CHUNK,), jnp.int32), pltpu.VMEM((CHUNK,), jnp.int32),
            pltpu.VMEM((CHUNK, 2 * EMBED), jnp.float32),
            pltpu.VMEM((CHUNK, 2 * EMBED), jnp.float32),
        ] + [pltpu.SemaphoreType.DMA] * 6,
    )
    def gather_kernel(idx_h, tab_h, out_h,
                      idx0, idx1, rb0, rb1,
                      nsem0, nsem1, gsem0, gsem1, osem0, osem1):
        wid = lax.axis_index("s") * NC + lax.axis_index("c")
        base = wid * PER_W

        slots = ((idx0, rb0, nsem0, gsem0, osem0),
                 (idx1, rb1, nsem1, gsem1, osem1))

        def issue_idx(g, slot):
            idxb, _, nsem, _, _ = slots[slot]
            pltpu.async_copy(idx_h.at[pl.ds(base + g * CHUNK, CHUNK)], idxb, nsem)

        def wait_idx(g, slot):
            idxb, _, nsem, _, _ = slots[slot]
            pltpu.make_async_copy(
                idx_h.at[pl.ds(base + g * CHUNK, CHUNK)], idxb, nsem).wait()

        def issue_gather(g, slot):
            idxb, rb, _, gsem, _ = slots[slot]
            pltpu.async_copy(tab_h.at[idxb], rb, gsem)

        def wait_gather(g, slot):
            idxb, rb, _, gsem, _ = slots[slot]
            pltpu.make_async_copy(tab_h.at[idxb], rb, gsem).wait()

        def issue_out(g, slot):
            _, rb, _, _, osem = slots[slot]
            pltpu.async_copy(rb, out_h.at[pl.ds(base + g * CHUNK, CHUNK)], osem)

        def wait_out(g, slot):
            _, rb, _, _, osem = slots[slot]
            pltpu.make_async_copy(
                rb, out_h.at[pl.ds(base + g * CHUNK, CHUNK)], osem).wait()

        issue_idx(0, 0)
        issue_idx(1, 1)
        wait_idx(0, 0)
        issue_gather(0, 0)
        wait_idx(1, 1)
        issue_gather(1, 1)

        def pair_body(gg, carry):
            g0 = 2 * gg
            g1 = g0 + 1

            def do(g, slot):
                wait_gather(g, slot)
                issue_out(g, slot)

                @pl.when(g + 2 < NCHUNK)
                def _():
                    issue_idx(g + 2, slot)
                    wait_out(g, slot)
                    wait_idx(g + 2, slot)
                    issue_gather(g + 2, slot)

                @pl.when(g + 2 >= NCHUNK)
                def _():
                    wait_out(g, slot)

            do(g0, 0)
            do(g1, 1)
            return carry

        lax.fori_loop(0, NCHUNK // 2, pair_body, 0)

    return gather_kernel


_sc_gather = _make_sc_gather()


def _tab_prep_body(t_ref, o_ref):
    t = jnp.transpose(t_ref[...], (1, 0))
    o_ref[:, :EMBED] = t


_tab_prep = pl.pallas_call(
    _tab_prep_body,
    grid=(VPAD // VBLK,),
    in_specs=[pl.BlockSpec((EMBED, VBLK), lambda j: (0, j))],
    out_specs=pl.BlockSpec((VBLK, 2 * EMBED), lambda j: (j, 0)),
    out_shape=jax.ShapeDtypeStruct((VPAD, 2 * EMBED), jnp.float32),
)


def _tc_add_body(x_ref, p_ref, v_ref, o_ref):
    for i in range(SBLK):
        vt = jnp.transpose(v_ref[i], (1, 0))
        o_ref[i] = x_ref[i] + p_ref[i] + vt[:EMBED]


def _make_tc_add():
    grid = (S // SBLK,)
    return pl.pallas_call(
        _tc_add_body,
        grid=grid,
        in_specs=[
            pl.BlockSpec((SBLK, EMBED, B), lambda s: (s, 0, 0)),
            pl.BlockSpec((SBLK, EMBED, B), lambda s: (s, 0, 0)),
            pl.BlockSpec((SBLK, B, 2 * EMBED), lambda s: (s, 0, 0)),
        ],
        out_specs=pl.BlockSpec((SBLK, EMBED, B), lambda s: (s, 0, 0)),
        out_shape=jax.ShapeDtypeStruct((S, EMBED, B), jnp.float32),
    )


_tc_add = _make_tc_add()


def kernel(x, variable_seq, pos_emb, table):
    idx_f = jnp.transpose(variable_seq, (1, 0)).reshape(N).astype(jnp.int32)
    tab_t = jnp.transpose(table, (1, 0))
    tab2 = _tab_prep(tab_t)
    var2 = _sc_gather(idx_f, tab2)
    x_t = jnp.transpose(x, (1, 2, 0))
    p_t = jnp.transpose(pos_emb, (1, 2, 0))
    o_t = _tc_add(x_t, p_t, var2.reshape(S, B, 2 * EMBED))
    return jnp.transpose(o_t, (2, 0, 1))
